# Initial kernel scaffold; baseline (speedup 1.0000x reference)
import jax
import jax.numpy as jnp
from jax.experimental import pallas as pl


def kernel(nodes, edges, senders, receivers, ew0, eb0, ew1, eb1, ew2, eb2, ew3, eb3, eg, ebeta, nw0, nb0, nw1, nb1, nw2, nb2, nw3, nb3, ng, nbeta):
    raise NotImplementedError("write your pallas kernel here")


# speedup vs baseline: 3.4915x; 3.4915x over previous
import functools

import jax
import jax.numpy as jnp
from jax import lax
from jax.experimental import pallas as pl
from jax.experimental.pallas import tpu as pltpu
from jax.experimental.pallas import tpu_sc as plsc

N = 10000
E = 320000
H = 128

_NC = 2
_NS = 16
_NW = _NC * _NS
_EPW = E // _NW
_C = 80
_NCH = _EPW // _C
_NPAD = 10240
_RPT = _NPAD // _NS

_F32 = jnp.float32


def _ln(x, g, b):
    mu = jnp.mean(x, axis=-1, keepdims=True)
    var = jnp.mean((x - mu) * (x - mu), axis=-1, keepdims=True)
    return (x - mu) * lax.rsqrt(var + 1e-5) * g + b


def _proj_body(n_ref, wa_ref, wb_ref, wc_ref, pa_ref, pb_ref, pc_ref):
    x = n_ref[...]
    pa_ref[...] = jnp.dot(x, wa_ref[...], preferred_element_type=_F32)
    pb_ref[...] = jnp.dot(x, wb_ref[...], preferred_element_type=_F32)
    pc_ref[...] = jnp.dot(x, wc_ref[...], preferred_element_type=_F32)


def _project_nodes(nodes, wa, wb, wc):
    bn = 2000
    grid = (N // bn,)
    blk = pl.BlockSpec((bn, H), lambda i: (i, 0))
    wblk = pl.BlockSpec((H, H), lambda i: (0, 0))
    out = jax.ShapeDtypeStruct((N, H), _F32)
    return pl.pallas_call(
        _proj_body,
        grid=grid,
        in_specs=[blk, wblk, wblk, wblk],
        out_specs=[blk, blk, blk],
        out_shape=[out, out, out],
    )(nodes, wa, wb, wc)


def _edge_body(gs_ref, gr_ref, e_ref, w0_ref, w1_ref, w2_ref, w3_ref,
               b0_ref, b1_ref, b2_ref, b3_ref, g_ref, bb_ref, o_ref):
    x = gs_ref[...] + gr_ref[...] + b0_ref[...]
    x = x + jnp.dot(e_ref[...], w0_ref[...], preferred_element_type=_F32)
    x = jnp.maximum(x, 0.0)
    x = jnp.maximum(jnp.dot(x, w1_ref[...], preferred_element_type=_F32) + b1_ref[...], 0.0)
    x = jnp.maximum(jnp.dot(x, w2_ref[...], preferred_element_type=_F32) + b2_ref[...], 0.0)
    x = jnp.dot(x, w3_ref[...], preferred_element_type=_F32) + b3_ref[...]
    o_ref[...] = _ln(x, g_ref[...], bb_ref[...])


def _edge_mlp(gs, gr, edges, w0c, w1, w2, w3, b0, b1, b2, b3, g, bb):
    be = 2000
    grid = (E // be,)
    blk = pl.BlockSpec((be, H), lambda i: (i, 0))
    wblk = pl.BlockSpec((H, H), lambda i: (0, 0))
    vblk = pl.BlockSpec((1, H), lambda i: (0, 0))
    return pl.pallas_call(
        _edge_body,
        grid=grid,
        in_specs=[blk, blk, blk, wblk, wblk, wblk, wblk,
                  vblk, vblk, vblk, vblk, vblk, vblk],
        out_specs=blk,
        out_shape=jax.ShapeDtypeStruct((E, H), _F32),
    )(gs, gr, edges, w0c, w1, w2, w3, b0, b1, b2, b3, g, bb)


def _node_body(q_ref, a0_ref, a1_ref, w0_ref, w1_ref, w2_ref, w3_ref,
               b0_ref, b1_ref, b2_ref, b3_ref, g_ref, bb_ref, o_ref):
    agg = a0_ref[...] + a1_ref[...]
    x = q_ref[...] + jnp.dot(agg, w0_ref[...], preferred_element_type=_F32) + b0_ref[...]
    x = jnp.maximum(x, 0.0)
    x = jnp.maximum(jnp.dot(x, w1_ref[...], preferred_element_type=_F32) + b1_ref[...], 0.0)
    x = jnp.maximum(jnp.dot(x, w2_ref[...], preferred_element_type=_F32) + b2_ref[...], 0.0)
    x = jnp.dot(x, w3_ref[...], preferred_element_type=_F32) + b3_ref[...]
    o_ref[...] = _ln(x, g_ref[...], bb_ref[...])


def _node_mlp(q, a0, a1, w0b, w1, w2, w3, b0, b1, b2, b3, g, bb):
    bn = 2000
    grid = (N // bn,)
    blk = pl.BlockSpec((bn, H), lambda i: (i, 0))
    wblk = pl.BlockSpec((H, H), lambda i: (0, 0))
    vblk = pl.BlockSpec((1, H), lambda i: (0, 0))
    return pl.pallas_call(
        _node_body,
        grid=grid,
        in_specs=[blk, blk, blk, wblk, wblk, wblk, wblk,
                  vblk, vblk, vblk, vblk, vblk, vblk],
        out_specs=blk,
        out_shape=jax.ShapeDtypeStruct((N, H), _F32),
    )(q, a0, a1, w0b, w1, w2, w3, b0, b1, b2, b3, g, bb)


def _sc_mesh():
    return plsc.VectorSubcoreMesh(core_axis_name="c", subcore_axis_name="s")


def _gather_kernel(ps_hbm, pr_hbm, s_hbm, r_hbm, gs_hbm, gr_hbm,
                   idx_s, idx_r, rows_s, rows_r, sem_s, sem_r):
    cid = lax.axis_index("c")
    sid = lax.axis_index("s")
    wid = sid * _NC + cid
    base = wid * _EPW
    pltpu.sync_copy(s_hbm.at[wid], idx_s)
    pltpu.sync_copy(r_hbm.at[wid], idx_r)

    @pl.loop(0, _NCH)
    def _chunk(j):
        off = base + j * _C
        cp_s = pltpu.async_copy(ps_hbm.at[idx_s.at[j]], rows_s, sem_s)
        cp_r = pltpu.async_copy(pr_hbm.at[idx_r.at[j]], rows_r, sem_r)
        cp_s.wait()
        pltpu.sync_copy(rows_s, gs_hbm.at[pl.ds(off, _C)])
        cp_r.wait()
        pltpu.sync_copy(rows_r, gr_hbm.at[pl.ds(off, _C)])


def _sc_gather(ps, pr, senders2d, receivers2d):
    out = jax.ShapeDtypeStruct((E, H), _F32)
    k = pl.kernel(
        _gather_kernel,
        out_type=(out, out),
        mesh=_sc_mesh(),
        scratch_types=[
            pltpu.VMEM((_NCH, _C), jnp.int32),
            pltpu.VMEM((_NCH, _C), jnp.int32),
            pltpu.VMEM((_C, H), _F32),
            pltpu.VMEM((_C, H), _F32),
            pltpu.SemaphoreType.DMA,
            pltpu.SemaphoreType.DMA,
        ],
    )
    return k(ps, pr, senders2d, receivers2d)


def _scatter_kernel(ne_hbm, r_hbm, zero_hbm, out_hbm, idx_v, rows_v, acc):
    cid = lax.axis_index("c")
    sid = lax.axis_index("s")
    wid = sid * _NC + cid
    base = wid * _EPW

    @pl.when(sid == 0)
    def _init():
        pltpu.sync_copy(zero_hbm, acc)

    plsc.subcore_barrier()
    pltpu.sync_copy(r_hbm.at[wid], idx_v)

    @pl.loop(0, _NCH)
    def _chunk(j):
        off = base + j * _C
        pltpu.sync_copy(ne_hbm.at[pl.ds(off, _C)], rows_v)
        pltpu.sync_copy(rows_v, acc.at[idx_v.at[j]], add=True)

    plsc.subcore_barrier()
    pltpu.sync_copy(acc.at[pl.ds(sid * _RPT, _RPT)],
                    out_hbm.at[cid, pl.ds(sid * _RPT, _RPT)])


def _sc_scatter(new_edges, receivers2d, zero):
    k = pl.kernel(
        _scatter_kernel,
        out_type=jax.ShapeDtypeStruct((_NC, _NPAD, H), _F32),
        mesh=_sc_mesh(),
        scratch_types=[
            pltpu.VMEM((_NCH, _C), jnp.int32),
            pltpu.VMEM((_C, H), _F32),
            pltpu.VMEM_SHARED((_NPAD, H), _F32),
        ],
    )
    return k(new_edges, receivers2d, zero)


def kernel(nodes, edges, senders, receivers,
           ew0, eb0, ew1, eb1, ew2, eb2, ew3, eb3, eg, ebeta,
           nw0, nb0, nw1, nb1, nw2, nb2, nw3, nb3, ng, nbeta):
    s2d = senders.astype(jnp.int32).reshape(_NW, _NCH, _C)
    r2d = receivers.astype(jnp.int32).reshape(_NW, _NCH, _C)

    ps, pr, qn = _project_nodes(nodes, ew0[:H], ew0[H:2 * H], nw0[:H])

    gs, gr = _sc_gather(ps, pr, s2d, r2d)

    row = lambda v: v.reshape(1, H)
    new_edges = _edge_mlp(gs, gr, edges, ew0[2 * H:], ew1, ew2, ew3,
                          row(eb0), row(eb1), row(eb2), row(eb3),
                          row(eg), row(ebeta))

    zero = jnp.zeros((_NPAD, H), _F32)
    partials = _sc_scatter(new_edges, r2d, zero)

    new_nodes = _node_mlp(qn, partials[0, :N], partials[1, :N], nw0[H:], nw1, nw2, nw3,
                          row(nb0), row(nb1), row(nb2), row(nb3),
                          row(ng), row(nbeta))
    return (new_nodes, new_edges)


# speedup vs baseline: 4.3278x; 1.2395x over previous
import functools

import jax
import jax.numpy as jnp
from jax import lax
from jax.experimental import pallas as pl
from jax.experimental.pallas import tpu as pltpu
from jax.experimental.pallas import tpu_sc as plsc

N = 10000
E = 320000
H = 128

_NC = 2
_NS = 16
_NW = _NC * _NS
_EPW = E // _NW
_C = 80
_NCH = _EPW // _C
_NPAD = 10240
_RPT = _NPAD // _NS

_F32 = jnp.float32


def _ln(x, g, b):
    mu = jnp.mean(x, axis=-1, keepdims=True)
    var = jnp.mean((x - mu) * (x - mu), axis=-1, keepdims=True)
    return (x - mu) * lax.rsqrt(var + 1e-5) * g + b


def _proj_body(n_ref, wa_ref, wb_ref, wc_ref, pa_ref, pb_ref, pc_ref):
    x = n_ref[...]
    pa_ref[...] = jnp.dot(x, wa_ref[...], preferred_element_type=_F32)
    pb_ref[...] = jnp.dot(x, wb_ref[...], preferred_element_type=_F32)
    pc_ref[...] = jnp.dot(x, wc_ref[...], preferred_element_type=_F32)


def _project_nodes(nodes, wa, wb, wc):
    bn = 2000
    grid = (N // bn,)
    blk = pl.BlockSpec((bn, H), lambda i: (i, 0))
    wblk = pl.BlockSpec((H, H), lambda i: (0, 0))
    out = jax.ShapeDtypeStruct((N, H), _F32)
    return pl.pallas_call(
        _proj_body,
        grid=grid,
        in_specs=[blk, wblk, wblk, wblk],
        out_specs=[blk, blk, blk],
        out_shape=[out, out, out],
    )(nodes, wa, wb, wc)


def _edge_body(gs_ref, gr_ref, e_ref, w0_ref, w1_ref, w2_ref, w3_ref,
               b0_ref, b1_ref, b2_ref, b3_ref, g_ref, bb_ref, o_ref):
    x = gs_ref[...] + gr_ref[...] + b0_ref[...]
    x = x + jnp.dot(e_ref[...], w0_ref[...], preferred_element_type=_F32)
    x = jnp.maximum(x, 0.0)
    x = jnp.maximum(jnp.dot(x, w1_ref[...], preferred_element_type=_F32) + b1_ref[...], 0.0)
    x = jnp.maximum(jnp.dot(x, w2_ref[...], preferred_element_type=_F32) + b2_ref[...], 0.0)
    x = jnp.dot(x, w3_ref[...], preferred_element_type=_F32) + b3_ref[...]
    o_ref[...] = _ln(x, g_ref[...], bb_ref[...])


def _edge_mlp(gs, gr, edges, w0c, w1, w2, w3, b0, b1, b2, b3, g, bb):
    be = 2000
    grid = (E // be,)
    blk = pl.BlockSpec((be, H), lambda i: (i, 0))
    wblk = pl.BlockSpec((H, H), lambda i: (0, 0))
    vblk = pl.BlockSpec((1, H), lambda i: (0, 0))
    return pl.pallas_call(
        _edge_body,
        grid=grid,
        in_specs=[blk, blk, blk, wblk, wblk, wblk, wblk,
                  vblk, vblk, vblk, vblk, vblk, vblk],
        out_specs=blk,
        out_shape=jax.ShapeDtypeStruct((E, H), _F32),
    )(gs, gr, edges, w0c, w1, w2, w3, b0, b1, b2, b3, g, bb)


def _node_body(q_ref, a0_ref, a1_ref, w0_ref, w1_ref, w2_ref, w3_ref,
               b0_ref, b1_ref, b2_ref, b3_ref, g_ref, bb_ref, o_ref):
    agg = a0_ref[0] + a1_ref[0]
    x = q_ref[...] + jnp.dot(agg, w0_ref[...], preferred_element_type=_F32) + b0_ref[...]
    x = jnp.maximum(x, 0.0)
    x = jnp.maximum(jnp.dot(x, w1_ref[...], preferred_element_type=_F32) + b1_ref[...], 0.0)
    x = jnp.maximum(jnp.dot(x, w2_ref[...], preferred_element_type=_F32) + b2_ref[...], 0.0)
    x = jnp.dot(x, w3_ref[...], preferred_element_type=_F32) + b3_ref[...]
    o_ref[...] = _ln(x, g_ref[...], bb_ref[...])


def _node_mlp(q, partials, w0b, w1, w2, w3, b0, b1, b2, b3, g, bb):
    bn = 2000
    grid = (N // bn,)
    blk = pl.BlockSpec((bn, H), lambda i: (i, 0))
    ablk0 = pl.BlockSpec((1, bn, H), lambda i: (0, i, 0))
    ablk1 = pl.BlockSpec((1, bn, H), lambda i: (1, i, 0))
    wblk = pl.BlockSpec((H, H), lambda i: (0, 0))
    vblk = pl.BlockSpec((1, H), lambda i: (0, 0))
    return pl.pallas_call(
        _node_body,
        grid=grid,
        in_specs=[blk, ablk0, ablk1, wblk, wblk, wblk, wblk,
                  vblk, vblk, vblk, vblk, vblk, vblk],
        out_specs=blk,
        out_shape=jax.ShapeDtypeStruct((N, H), _F32),
    )(q, partials, partials, w0b, w1, w2, w3, b0, b1, b2, b3, g, bb)


def _sc_mesh():
    return plsc.VectorSubcoreMesh(core_axis_name="c", subcore_axis_name="s")


def _gather_kernel(ps_hbm, pr_hbm, s_hbm, r_hbm, gs_hbm, gr_hbm, *scr):
    idx = {"s": scr[0], "r": scr[1]}
    rows = {("s", 0): scr[2], ("s", 1): scr[3], ("r", 0): scr[4], ("r", 1): scr[5]}
    gsem = {("s", 0): scr[6], ("s", 1): scr[7], ("r", 0): scr[8], ("r", 1): scr[9]}
    wsem = {("s", 0): scr[10], ("s", 1): scr[11], ("r", 0): scr[12], ("r", 1): scr[13]}
    tab = {"s": ps_hbm, "r": pr_hbm}
    out = {"s": gs_hbm, "r": gr_hbm}

    cid = lax.axis_index("c")
    sid = lax.axis_index("s")
    wid = sid * _NC + cid
    base = wid * _EPW
    pltpu.sync_copy(s_hbm.at[wid], idx["s"])
    pltpu.sync_copy(r_hbm.at[wid], idx["r"])

    def issue_g(c, b):
        for t in ("s", "r"):
            pltpu.async_copy(tab[t].at[idx[t].at[c]], rows[(t, b)], gsem[(t, b)])

    def wait_g(b):
        for t in ("s", "r"):
            pltpu.make_async_copy(tab[t].at[pl.ds(0, _C)], rows[(t, b)],
                                  gsem[(t, b)]).wait()

    def start_wb(c, b):
        for t in ("s", "r"):
            pltpu.async_copy(rows[(t, b)], out[t].at[pl.ds(base + c * _C, _C)],
                             wsem[(t, b)])

    def wait_wb(b):
        for t in ("s", "r"):
            pltpu.make_async_copy(rows[(t, b)], out[t].at[pl.ds(0, _C)],
                                  wsem[(t, b)]).wait()

    issue_g(0, 0)

    @pl.loop(0, (_NCH - 1) // 2)
    def _pair(j):
        c = 2 * j

        @pl.when(j > 0)
        def _():
            wait_wb(1)

        issue_g(c + 1, 1)
        wait_g(0)
        start_wb(c, 0)
        wait_wb(0)
        issue_g(c + 2, 0)
        wait_g(1)
        start_wb(c + 1, 1)

    wait_wb(1)
    wait_g(0)
    start_wb(_NCH - 1, 0)
    wait_wb(0)


def _sc_gather(ps, pr, senders2d, receivers2d):
    out = jax.ShapeDtypeStruct((E, H), _F32)
    rowbuf = pltpu.VMEM((_C, H), _F32)
    k = pl.kernel(
        _gather_kernel,
        out_type=(out, out),
        mesh=_sc_mesh(),
        scratch_types=(
            [pltpu.VMEM((_NCH, _C), jnp.int32)] * 2
            + [rowbuf] * 4
            + [pltpu.SemaphoreType.DMA] * 8
        ),
    )
    return k(ps, pr, senders2d, receivers2d)


def _scatter_kernel(ne_hbm, r_hbm, zero_hbm, out_hbm, *scr):
    idx_v = scr[0]
    rows = (scr[1], scr[2])
    acc = scr[3]
    lsem = (scr[4], scr[5])
    ssem = (scr[6], scr[7])

    cid = lax.axis_index("c")
    sid = lax.axis_index("s")
    wid = sid * _NC + cid
    base = wid * _EPW

    @pl.when(sid == 0)
    def _init():
        pltpu.sync_copy(zero_hbm, acc)

    pltpu.sync_copy(r_hbm.at[wid], idx_v)
    plsc.subcore_barrier()

    def load(c, b):
        pltpu.async_copy(ne_hbm.at[pl.ds(base + c * _C, _C)], rows[b], lsem[b])

    def wait_load(b):
        pltpu.make_async_copy(ne_hbm.at[pl.ds(0, _C)], rows[b], lsem[b]).wait()

    def start_sc(c, b):
        pltpu.async_copy(rows[b], acc.at[idx_v.at[c]], ssem[b], add=True)

    def wait_sc(b):
        pltpu.make_async_copy(rows[b], acc.at[idx_v.at[0]], ssem[b]).wait()

    load(0, 0)

    @pl.loop(0, (_NCH - 1) // 2)
    def _pair(j):
        c = 2 * j

        @pl.when(j > 0)
        def _():
            wait_sc(1)

        load(c + 1, 1)
        wait_load(0)
        start_sc(c, 0)
        wait_sc(0)
        load(c + 2, 0)
        wait_load(1)
        start_sc(c + 1, 1)

    wait_sc(1)
    wait_load(0)
    start_sc(_NCH - 1, 0)
    wait_sc(0)

    plsc.subcore_barrier()
    pltpu.sync_copy(acc.at[pl.ds(sid * _RPT, _RPT)],
                    out_hbm.at[cid, pl.ds(sid * _RPT, _RPT)])


def _sc_scatter(new_edges, receivers2d, zero):
    k = pl.kernel(
        _scatter_kernel,
        out_type=jax.ShapeDtypeStruct((_NC, _NPAD, H), _F32),
        mesh=_sc_mesh(),
        scratch_types=(
            [pltpu.VMEM((_NCH, _C), jnp.int32)]
            + [pltpu.VMEM((_C, H), _F32)] * 2
            + [pltpu.VMEM_SHARED((_NPAD, H), _F32)]
            + [pltpu.SemaphoreType.DMA] * 4
        ),
    )
    return k(new_edges, receivers2d, zero)


def kernel(nodes, edges, senders, receivers,
           ew0, eb0, ew1, eb1, ew2, eb2, ew3, eb3, eg, ebeta,
           nw0, nb0, nw1, nb1, nw2, nb2, nw3, nb3, ng, nbeta):
    s2d = senders.astype(jnp.int32).reshape(_NW, _NCH, _C)
    r2d = receivers.astype(jnp.int32).reshape(_NW, _NCH, _C)

    ps, pr, qn = _project_nodes(nodes, ew0[:H], ew0[H:2 * H], nw0[:H])

    gs, gr = _sc_gather(ps, pr, s2d, r2d)

    row = lambda v: v.reshape(1, H)
    new_edges = _edge_mlp(gs, gr, edges, ew0[2 * H:], ew1, ew2, ew3,
                          row(eb0), row(eb1), row(eb2), row(eb3),
                          row(eg), row(ebeta))

    zero = jnp.zeros((_NPAD, H), _F32)
    partials = _sc_scatter(new_edges, r2d, zero)

    new_nodes = _node_mlp(qn, partials, nw0[H:], nw1, nw2, nw3,
                          row(nb0), row(nb1), row(nb2), row(nb3),
                          row(ng), row(nbeta))
    return (new_nodes, new_edges)


# speedup vs baseline: 4.8360x; 1.1174x over previous
import functools

import jax
import jax.numpy as jnp
from jax import lax
from jax.experimental import pallas as pl
from jax.experimental.pallas import tpu as pltpu
from jax.experimental.pallas import tpu_sc as plsc

N = 10000
E = 320000
H = 128

_NC = 2
_NS = 16
_NW = _NC * _NS
_EPW = E // _NW
_C = 80
_NCH = _EPW // _C
_EPT = E // _NS
_GCH = _EPT // _C
_GSTG = 50
_NPAD = 10240
_RPT = _NPAD // _NS

_F32 = jnp.float32


def _ln(x, g, b):
    mu = jnp.mean(x, axis=-1, keepdims=True)
    var = jnp.mean((x - mu) * (x - mu), axis=-1, keepdims=True)
    return (x - mu) * lax.rsqrt(var + 1e-5) * g + b


def _proj_body(n_ref, wa_ref, wb_ref, wc_ref, psr_ref, pc_ref):
    x = n_ref[...]
    psr_ref[0] = jnp.dot(x, wa_ref[...], preferred_element_type=_F32)
    psr_ref[1] = jnp.dot(x, wb_ref[...], preferred_element_type=_F32)
    pc_ref[...] = jnp.dot(x, wc_ref[...], preferred_element_type=_F32)


def _project_nodes(nodes_pad, wa, wb, wc):
    bn = 2048
    grid = (_NPAD // bn,)
    blk = pl.BlockSpec((bn, H), lambda i: (i, 0))
    wblk = pl.BlockSpec((H, H), lambda i: (0, 0))
    return pl.pallas_call(
        _proj_body,
        grid=grid,
        in_specs=[blk, wblk, wblk, wblk],
        out_specs=[pl.BlockSpec((2, bn, H), lambda i: (0, i, 0)), blk],
        out_shape=[jax.ShapeDtypeStruct((2, _NPAD, H), _F32),
                   jax.ShapeDtypeStruct((_NPAD, H), _F32)],
    )(nodes_pad, wa, wb, wc)


def _edge_body(gs_ref, gr_ref, e_ref, w0_ref, w1_ref, w2_ref, w3_ref,
               b0_ref, b1_ref, b2_ref, b3_ref, g_ref, bb_ref, o_ref):
    x = gs_ref[0] + gr_ref[0] + b0_ref[...]
    x = x + jnp.dot(e_ref[...], w0_ref[...], preferred_element_type=_F32)
    x = jnp.maximum(x, 0.0)
    x = jnp.maximum(jnp.dot(x, w1_ref[...], preferred_element_type=_F32) + b1_ref[...], 0.0)
    x = jnp.maximum(jnp.dot(x, w2_ref[...], preferred_element_type=_F32) + b2_ref[...], 0.0)
    x = jnp.dot(x, w3_ref[...], preferred_element_type=_F32) + b3_ref[...]
    o_ref[...] = _ln(x, g_ref[...], bb_ref[...])


def _edge_mlp(gsr, edges, w0c, w1, w2, w3, b0, b1, b2, b3, g, bb):
    be = 2000
    grid = (E // be,)
    blk = pl.BlockSpec((be, H), lambda i: (i, 0))
    gblk0 = pl.BlockSpec((1, be, H), lambda i: (0, i, 0))
    gblk1 = pl.BlockSpec((1, be, H), lambda i: (1, i, 0))
    wblk = pl.BlockSpec((H, H), lambda i: (0, 0))
    vblk = pl.BlockSpec((1, H), lambda i: (0, 0))
    return pl.pallas_call(
        _edge_body,
        grid=grid,
        in_specs=[gblk0, gblk1, blk, wblk, wblk, wblk, wblk,
                  vblk, vblk, vblk, vblk, vblk, vblk],
        out_specs=blk,
        out_shape=jax.ShapeDtypeStruct((E, H), _F32),
    )(gsr, gsr, edges, w0c, w1, w2, w3, b0, b1, b2, b3, g, bb)


def _node_body(q_ref, a0_ref, a1_ref, w0_ref, w1_ref, w2_ref, w3_ref,
               b0_ref, b1_ref, b2_ref, b3_ref, g_ref, bb_ref, o_ref):
    agg = a0_ref[0] + a1_ref[0]
    x = q_ref[...] + jnp.dot(agg, w0_ref[...], preferred_element_type=_F32) + b0_ref[...]
    x = jnp.maximum(x, 0.0)
    x = jnp.maximum(jnp.dot(x, w1_ref[...], preferred_element_type=_F32) + b1_ref[...], 0.0)
    x = jnp.maximum(jnp.dot(x, w2_ref[...], preferred_element_type=_F32) + b2_ref[...], 0.0)
    x = jnp.dot(x, w3_ref[...], preferred_element_type=_F32) + b3_ref[...]
    o_ref[...] = _ln(x, g_ref[...], bb_ref[...])


def _node_mlp(q, partials, w0b, w1, w2, w3, b0, b1, b2, b3, g, bb):
    bn = 2000
    grid = (N // bn,)
    blk = pl.BlockSpec((bn, H), lambda i: (i, 0))
    ablk0 = pl.BlockSpec((1, bn, H), lambda i: (0, i, 0))
    ablk1 = pl.BlockSpec((1, bn, H), lambda i: (1, i, 0))
    wblk = pl.BlockSpec((H, H), lambda i: (0, 0))
    vblk = pl.BlockSpec((1, H), lambda i: (0, 0))
    return pl.pallas_call(
        _node_body,
        grid=grid,
        in_specs=[blk, ablk0, ablk1, wblk, wblk, wblk, wblk,
                  vblk, vblk, vblk, vblk, vblk, vblk],
        out_specs=blk,
        out_shape=jax.ShapeDtypeStruct((N, H), _F32),
    )(q, partials, partials, w0b, w1, w2, w3, b0, b1, b2, b3, g, bb)


def _sc_mesh():
    return plsc.VectorSubcoreMesh(core_axis_name="c", subcore_axis_name="s")


def _pipeline(nch, issue, wait_fill, start_drain, wait_drain):
    npairs = (nch - 1) // 2
    issue(0, 0)

    @pl.loop(0, npairs)
    def _pair(j):
        c = 2 * j

        @pl.when(j > 0)
        def _():
            wait_drain(1)

        issue(c + 1, 1)
        wait_fill(0)
        start_drain(c, 0)
        wait_drain(0)
        issue(c + 2, 0)
        wait_fill(1)
        start_drain(c + 1, 1)

    wait_drain(1)
    wait_fill(0)
    start_drain(2 * npairs, 0)
    if nch % 2 == 0:
        issue(nch - 1, 1)
        wait_drain(0)
        wait_fill(1)
        start_drain(nch - 1, 1)
        wait_drain(1)
    else:
        wait_drain(0)


def _gather_kernel(psr_hbm, srx_hbm, out_hbm, idx_v, tabs, r0, r1, g0, g1, w0, w1):
    cid = lax.axis_index("c")
    sid = lax.axis_index("s")
    rows = (r0, r1)
    gsem = (g0, g1)
    wsem = (w0, w1)
    base = sid * _EPT

    pltpu.sync_copy(psr_hbm.at[cid, pl.ds(sid * _RPT, _RPT)],
                    tabs.at[pl.ds(sid * _RPT, _RPT)])
    plsc.subcore_barrier()

    def issue(c, b):
        pltpu.async_copy(tabs.at[idx_v.at[c]], rows[b], gsem[b])

    def wait_fill(b):
        pltpu.make_async_copy(tabs.at[pl.ds(0, _C)], rows[b], gsem[b]).wait()

    ngrp = _GCH // _GSTG
    for g in range(ngrp):
        gb = base + g * _GSTG * _C
        pltpu.sync_copy(srx_hbm.at[(cid * _NS + sid) * ngrp + g], idx_v)

        def start_drain(c, b, gb=gb):
            pltpu.async_copy(rows[b], out_hbm.at[cid, pl.ds(gb + c * _C, _C)],
                             wsem[b])

        def wait_drain(b):
            pltpu.make_async_copy(rows[b], out_hbm.at[cid, pl.ds(0, _C)],
                                  wsem[b]).wait()

        _pipeline(_GSTG, issue, wait_fill, start_drain, wait_drain)


def _sc_gather(psr, srx):
    k = pl.kernel(
        _gather_kernel,
        out_type=jax.ShapeDtypeStruct((2, E, H), _F32),
        mesh=_sc_mesh(),
        scratch_types=(
            [pltpu.VMEM((_GSTG, _C), jnp.int32)]
            + [pltpu.VMEM_SHARED((_NPAD, H), _F32)]
            + [pltpu.VMEM((_C, H), _F32)] * 2
            + [pltpu.SemaphoreType.DMA] * 4
        ),
    )
    return k(psr, srx)


def _scatter_kernel(ne_hbm, r_hbm, zero_hbm, out_hbm, *scr):
    idx_v = scr[0]
    rows = (scr[1], scr[2])
    acc = scr[3]
    lsem = (scr[4], scr[5])
    ssem = (scr[6], scr[7])

    cid = lax.axis_index("c")
    sid = lax.axis_index("s")
    wid = sid * _NC + cid
    base = wid * _EPW

    @pl.when(sid == 0)
    def _init():
        pltpu.sync_copy(zero_hbm, acc)

    pltpu.sync_copy(r_hbm.at[wid], idx_v)
    plsc.subcore_barrier()

    def load(c, b):
        pltpu.async_copy(ne_hbm.at[pl.ds(base + c * _C, _C)], rows[b], lsem[b])

    def wait_load(b):
        pltpu.make_async_copy(ne_hbm.at[pl.ds(0, _C)], rows[b], lsem[b]).wait()

    def start_sc(c, b):
        pltpu.async_copy(rows[b], acc.at[idx_v.at[c]], ssem[b], add=True)

    def wait_sc(b):
        pltpu.make_async_copy(rows[b], acc.at[idx_v.at[0]], ssem[b]).wait()

    _pipeline(_NCH, load, wait_load, start_sc, wait_sc)

    plsc.subcore_barrier()
    pltpu.sync_copy(acc.at[pl.ds(sid * _RPT, _RPT)],
                    out_hbm.at[cid, pl.ds(sid * _RPT, _RPT)])


def _sc_scatter(new_edges, receivers2d, zero):
    k = pl.kernel(
        _scatter_kernel,
        out_type=jax.ShapeDtypeStruct((_NC, _NPAD, H), _F32),
        mesh=_sc_mesh(),
        scratch_types=(
            [pltpu.VMEM((_NCH, _C), jnp.int32)]
            + [pltpu.VMEM((_C, H), _F32)] * 2
            + [pltpu.VMEM_SHARED((_NPAD, H), _F32)]
            + [pltpu.SemaphoreType.DMA] * 4
        ),
    )
    return k(new_edges, receivers2d, zero)


def kernel(nodes, edges, senders, receivers,
           ew0, eb0, ew1, eb1, ew2, eb2, ew3, eb3, eg, ebeta,
           nw0, nb0, nw1, nb1, nw2, nb2, nw3, nb3, ng, nbeta):
    s32 = senders.astype(jnp.int32)
    r32 = receivers.astype(jnp.int32)
    srx = jnp.stack([s32, r32]).reshape(-1, _GSTG, _C)
    r2d = r32.reshape(_NW, _NCH, _C)

    nodes_pad = jnp.pad(nodes, ((0, _NPAD - N), (0, 0)))
    psr, qn = _project_nodes(nodes_pad, ew0[:H], ew0[H:2 * H], nw0[:H])

    gsr = _sc_gather(psr, srx)

    row = lambda v: v.reshape(1, H)
    new_edges = _edge_mlp(gsr, edges, ew0[2 * H:], ew1, ew2, ew3,
                          row(eb0), row(eb1), row(eb2), row(eb3),
                          row(eg), row(ebeta))

    zero = jnp.zeros((_NPAD, H), _F32)
    partials = _sc_scatter(new_edges, r2d, zero)

    new_nodes = _node_mlp(qn, partials, nw0[H:], nw1, nw2, nw3,
                          row(nb0), row(nb1), row(nb2), row(nb3),
                          row(ng), row(nbeta))
    return (new_nodes, new_edges)
